# baseline (speedup 1.0000x reference)
import functools

import jax
import jax.numpy as jnp
from jax import lax
from jax.experimental import pallas as pl
from jax.experimental.pallas import tpu as pltpu
from jax.experimental.pallas import tpu_sc as plsc

POOL_H = 7
POOL_W = 7
NUM_ROI_GROUPS = 8
NUM_C_GROUPS = 4
C_PER_TILE = 16
NB = 25


def _make_roi_kernel(H, W, R, C):
    rois_per_tile = R // NUM_ROI_GROUPS
    n_batches = rois_per_tile // NB
    mesh = plsc.VectorSubcoreMesh(core_axis_name="c", subcore_axis_name="s")

    @functools.partial(
        pl.kernel,
        out_type=jax.ShapeDtypeStruct(
            (R, POOL_H, POOL_W, NUM_C_GROUPS, C_PER_TILE), jnp.float32),
        mesh=mesh,
        compiler_params=pltpu.CompilerParams(use_tc_tiling_on_sc=False),
        scratch_types=[
            pltpu.VMEM((H, W, C_PER_TILE), jnp.float32),
            pltpu.VMEM((rois_per_tile, 16), jnp.int32),
            pltpu.VMEM((2, NB, POOL_H, POOL_W, C_PER_TILE), jnp.float32),
            pltpu.SemaphoreType.DMA,
            pltpu.SemaphoreType.DMA,
        ],
    )
    def roi_kernel(fm_hbm, rois_hbm, out_hbm, fm_v, roi_v, out_buf,
                   sem0, sem1):
        wid = lax.axis_index("s") * 2 + lax.axis_index("c")
        c_g = wid // NUM_ROI_GROUPS
        roi_g = wid % NUM_ROI_GROUPS
        roi_base = roi_g * rois_per_tile

        pltpu.sync_copy(fm_hbm.at[c_g], fm_v)
        pltpu.sync_copy(rois_hbm.at[roi_g], roi_v)

        lane = lax.iota(jnp.int32, 16)
        pos = lane & 7
        posf = pos.astype(jnp.float32) + 0.5
        zeros = lane & 0
        idx_base = 1 - (lane >> 3)
        idx_end = idx_base + 2
        gather_dnums = lax.GatherDimensionNumbers(
            offset_dims=(), collapsed_slice_dims=(0,), start_index_map=(0,))

        def vgather(x, idx):
            return lax.gather(
                x, idx[:, None], gather_dnums, (1,),
                mode=lax.GatherScatterMode.PROMISE_IN_BOUNDS)

        def fill_one(j, slot, jj):
            rv = roi_v[j]
            base_s = vgather(rv, idx_base)
            ends = vgather(rv, idx_end)
            cl = ends - base_s
            clf = cl.astype(jnp.float32)
            coord = posf * (clf / float(POOL_H)) - 0.5
            coord = jnp.clip(coord, 0.0, jnp.maximum(clf - 1.0, 0.0))
            f0 = coord.astype(jnp.int32)
            w = coord - f0.astype(jnp.float32)
            a = base_s + f0
            y2m1 = rv[3] - 1
            x2m1 = rv[2] - 1
            ya = [a[p] for p in range(POOL_H)]
            xa = [a[8 + q] for q in range(POOL_W)]
            yb = [jnp.minimum(ya[p] + 1, y2m1) for p in range(POOL_H)]
            xb = [jnp.minimum(xa[q] + 1, x2m1) for q in range(POOL_W)]
            wy = [vgather(w, zeros + p) for p in range(POOL_H)]
            wx = [vgather(w, zeros + (8 + q)) for q in range(POOL_W)]
            for p in range(POOL_H):
                for q in range(POOL_W):
                    g_aa = fm_v[ya[p], xa[q]]
                    g_ab = fm_v[ya[p], xb[q]]
                    g_ba = fm_v[yb[p], xa[q]]
                    g_bb = fm_v[yb[p], xb[q]]
                    top = g_aa + wx[q] * (g_ab - g_aa)
                    bot = g_ba + wx[q] * (g_bb - g_ba)
                    out_buf[slot, jj, p, q] = top + wy[p] * (bot - top)

        def out_slice(bi):
            return out_hbm.at[pl.ds(roi_base + bi * NB, NB), :, :, c_g]

        def batch_body(bi, _):
            slot = bi & 1

            @pl.when((bi >= 2) & (slot == 0))
            def _():
                pltpu.make_async_copy(
                    out_buf.at[0], out_slice(bi - 2), sem0).wait()

            @pl.when((bi >= 2) & (slot == 1))
            def _():
                pltpu.make_async_copy(
                    out_buf.at[1], out_slice(bi - 2), sem1).wait()

            @plsc.parallel_loop(0, NB, unroll=4)
            def _(jj):
                fill_one(bi * NB + jj, slot, jj)

            @pl.when(slot == 0)
            def _():
                pltpu.make_async_copy(
                    out_buf.at[0], out_slice(bi), sem0).start()

            @pl.when(slot == 1)
            def _():
                pltpu.make_async_copy(
                    out_buf.at[1], out_slice(bi), sem1).start()

            return 0

        lax.fori_loop(0, n_batches, batch_body, 0)

        pltpu.make_async_copy(
            out_buf.at[0], out_slice(n_batches - 2), sem0).wait()
        pltpu.make_async_copy(
            out_buf.at[1], out_slice(n_batches - 1), sem1).wait()

    return roi_kernel


@jax.jit
def kernel(feature_map, rois):
    _, H, W, C = feature_map.shape
    _, R, _ = rois.shape
    fm_t = feature_map[0].reshape(H, W, NUM_C_GROUPS, C_PER_TILE)
    fm_t = jnp.transpose(fm_t, (2, 0, 1, 3))
    rois_t = rois[0].reshape(NUM_ROI_GROUPS, R // NUM_ROI_GROUPS, 4)
    rois_t = jnp.pad(rois_t, ((0, 0), (0, 0), (0, 12)))
    out = _make_roi_kernel(H, W, R, C)(fm_t, rois_t)
    return out.reshape(1, R, POOL_H, POOL_W, C)


# speedup vs baseline: 1.3386x; 1.3386x over previous
import functools

import jax
import jax.numpy as jnp
from jax import lax
from jax.experimental import pallas as pl
from jax.experimental.pallas import tpu as pltpu
from jax.experimental.pallas import tpu_sc as plsc

POOL_H = 7
POOL_W = 7
NUM_ROI_GROUPS = 8
NUM_C_GROUPS = 4
C_PER_TILE = 16
NB = 25


def _make_roi_kernel(H, W, R, C):
    rois_per_tile = R // NUM_ROI_GROUPS
    n_batches = rois_per_tile // NB
    mesh = plsc.VectorSubcoreMesh(core_axis_name="c", subcore_axis_name="s")

    @functools.partial(
        pl.kernel,
        out_type=jax.ShapeDtypeStruct(
            (R, POOL_H, POOL_W, NUM_C_GROUPS, C_PER_TILE), jnp.float32),
        mesh=mesh,
        compiler_params=pltpu.CompilerParams(use_tc_tiling_on_sc=False),
        scratch_types=[
            pltpu.VMEM((H, W, C_PER_TILE), jnp.float32),
            pltpu.VMEM((rois_per_tile, 16), jnp.int32),
            pltpu.VMEM((2, NB, POOL_H, POOL_W, C_PER_TILE), jnp.float32),
            pltpu.SemaphoreType.DMA,
            pltpu.SemaphoreType.DMA,
        ],
    )
    def roi_kernel(fm_hbm, rois_hbm, out_hbm, fm_v, roi_v, out_buf,
                   sem0, sem1):
        wid = lax.axis_index("s") * 2 + lax.axis_index("c")
        c_g = wid // NUM_ROI_GROUPS
        roi_g = wid % NUM_ROI_GROUPS
        roi_base = roi_g * rois_per_tile

        pltpu.sync_copy(fm_hbm.at[c_g], fm_v)
        pltpu.sync_copy(rois_hbm.at[roi_g], roi_v)

        lane = lax.iota(jnp.int32, 16)
        pos = lane & 7
        posf = pos.astype(jnp.float32) + 0.5
        zeros = lane & 0
        idx_base = 1 - (lane >> 3)
        idx_end = idx_base + 2
        gather_dnums = lax.GatherDimensionNumbers(
            offset_dims=(), collapsed_slice_dims=(0,), start_index_map=(0,))

        def vgather(x, idx):
            return lax.gather(
                x, idx[:, None], gather_dnums, (1,),
                mode=lax.GatherScatterMode.PROMISE_IN_BOUNDS)

        def fill_one(j, slot, jj):
            rv = roi_v[j]
            base_s = vgather(rv, idx_base)
            ends = vgather(rv, idx_end)
            cl = ends - base_s
            clf = cl.astype(jnp.float32)
            coord = posf * (clf / float(POOL_H)) - 0.5
            coord = jnp.clip(coord, 0.0, jnp.maximum(clf - 1.0, 0.0))
            f0 = coord.astype(jnp.int32)
            w = coord - f0.astype(jnp.float32)
            a = base_s + f0
            y2m1 = rv[3] - 1
            x2m1 = rv[2] - 1
            ya = [a[p] for p in range(POOL_H)]
            xa = [a[8 + q] for q in range(POOL_W)]
            yb = [jnp.minimum(ya[p] + 1, y2m1) for p in range(POOL_H)]
            xb = [jnp.minimum(xa[q] + 1, x2m1) for q in range(POOL_W)]
            wy = [vgather(w, zeros + p) for p in range(POOL_H)]
            wx = [vgather(w, zeros + (8 + q)) for q in range(POOL_W)]
            for p in range(POOL_H):
                for q in range(POOL_W):
                    g_aa = fm_v[ya[p], xa[q]]
                    g_ab = fm_v[ya[p], xb[q]]
                    g_ba = fm_v[yb[p], xa[q]]
                    g_bb = fm_v[yb[p], xb[q]]
                    top = g_aa + wx[q] * (g_ab - g_aa)
                    bot = g_ba + wx[q] * (g_bb - g_ba)
                    out_buf[slot, jj, p, q] = top + wy[p] * (bot - top)

        def out_slice(bi):
            return out_hbm.at[pl.ds(roi_base + bi * NB, NB), :, :, c_g]

        def batch_body(bi, _):
            slot = bi & 1

            @pl.when((bi >= 2) & (slot == 0))
            def _():
                pltpu.make_async_copy(
                    out_buf.at[0], out_slice(bi - 2), sem0).wait()

            @pl.when((bi >= 2) & (slot == 1))
            def _():
                pltpu.make_async_copy(
                    out_buf.at[1], out_slice(bi - 2), sem1).wait()

            @plsc.parallel_loop(0, NB, unroll=2)
            def _(jj):
                fill_one(bi * NB + jj, slot, jj)

            @pl.when(slot == 0)
            def _():
                pltpu.make_async_copy(
                    out_buf.at[0], out_slice(bi), sem0).start()

            @pl.when(slot == 1)
            def _():
                pltpu.make_async_copy(
                    out_buf.at[1], out_slice(bi), sem1).start()

            return 0

        lax.fori_loop(0, n_batches, batch_body, 0)

        pltpu.make_async_copy(
            out_buf.at[0], out_slice(n_batches - 2), sem0).wait()
        pltpu.make_async_copy(
            out_buf.at[1], out_slice(n_batches - 1), sem1).wait()

    return roi_kernel


@jax.jit
def kernel(feature_map, rois):
    _, H, W, C = feature_map.shape
    _, R, _ = rois.shape
    fm_t = feature_map[0].reshape(H, W, NUM_C_GROUPS, C_PER_TILE)
    fm_t = jnp.transpose(fm_t, (2, 0, 1, 3))
    rois_t = rois[0].reshape(NUM_ROI_GROUPS, R // NUM_ROI_GROUPS, 4)
    rois_t = jnp.pad(rois_t, ((0, 0), (0, 0), (0, 12)))
    out = _make_roi_kernel(H, W, R, C)(fm_t, rois_t)
    return out.reshape(1, R, POOL_H, POOL_W, C)


# speedup vs baseline: 1.7185x; 1.2837x over previous
import functools

import jax
import jax.numpy as jnp
from jax import lax
from jax.experimental import pallas as pl
from jax.experimental.pallas import tpu as pltpu
from jax.experimental.pallas import tpu_sc as plsc

POOL_H = 7
POOL_W = 7
NUM_ROI_GROUPS = 8
NUM_C_GROUPS = 4
C_PER_TILE = 16
NB = 25


def _make_roi_kernel(H, W, R, C):
    rois_per_tile = R // NUM_ROI_GROUPS
    n_batches = rois_per_tile // NB
    mesh = plsc.VectorSubcoreMesh(core_axis_name="c", subcore_axis_name="s")

    @functools.partial(
        pl.kernel,
        out_type=jax.ShapeDtypeStruct(
            (R, POOL_H, POOL_W, NUM_C_GROUPS, C_PER_TILE), jnp.float32),
        mesh=mesh,
        compiler_params=pltpu.CompilerParams(use_tc_tiling_on_sc=False),
        scratch_types=[
            pltpu.VMEM((H, W, C_PER_TILE), jnp.float32),
            pltpu.VMEM((rois_per_tile, 16), jnp.int32),
            pltpu.VMEM((2, NB, POOL_H, POOL_W, C_PER_TILE), jnp.float32),
            pltpu.SemaphoreType.DMA,
            pltpu.SemaphoreType.DMA,
        ],
    )
    def roi_kernel(fm_hbm, rois_hbm, out_hbm, fm_v, roi_v, out_buf,
                   sem0, sem1):
        wid = lax.axis_index("s") * 2 + lax.axis_index("c")
        c_g = wid // NUM_ROI_GROUPS
        roi_g = wid % NUM_ROI_GROUPS
        roi_base = roi_g * rois_per_tile

        pltpu.sync_copy(fm_hbm.at[c_g], fm_v)
        pltpu.sync_copy(rois_hbm.at[roi_g], roi_v)

        lane = lax.iota(jnp.int32, 16)
        pos = lane & 7
        posf = pos.astype(jnp.float32) + 0.5
        zeros = lane & 0
        idx_base = 1 - (lane >> 3)
        idx_end = idx_base + 2
        gather_dnums = lax.GatherDimensionNumbers(
            offset_dims=(), collapsed_slice_dims=(0,), start_index_map=(0,))

        def vgather(x, idx):
            return lax.gather(
                x, idx[:, None], gather_dnums, (1,),
                mode=lax.GatherScatterMode.PROMISE_IN_BOUNDS)

        def fill_one(j, slot, jj):
            rv = roi_v[j]
            base_s = vgather(rv, idx_base)
            ends = vgather(rv, idx_end)
            cl = ends - base_s
            clf = cl.astype(jnp.float32)
            coord = posf * (clf / float(POOL_H)) - 0.5
            coord = jnp.clip(coord, 0.0, jnp.maximum(clf - 1.0, 0.0))
            f0 = coord.astype(jnp.int32)
            w = coord - f0.astype(jnp.float32)
            a = base_s + f0
            y2m1 = rv[3] - 1
            x2m1 = rv[2] - 1
            ya = [a[p] for p in range(POOL_H)]
            xa = [a[8 + q] for q in range(POOL_W)]
            yb = [jnp.minimum(ya[p] + 1, y2m1) for p in range(POOL_H)]
            xb = [jnp.minimum(xa[q] + 1, x2m1) for q in range(POOL_W)]
            wy = [vgather(w, zeros + p) for p in range(POOL_H)]
            wx = [vgather(w, zeros + (8 + q)) for q in range(POOL_W)]
            for p in range(POOL_H):
                for q in range(POOL_W):
                    g_aa = fm_v[ya[p], xa[q]]
                    g_ab = fm_v[ya[p], xb[q]]
                    g_ba = fm_v[yb[p], xa[q]]
                    g_bb = fm_v[yb[p], xb[q]]
                    top = g_aa + wx[q] * (g_ab - g_aa)
                    bot = g_ba + wx[q] * (g_bb - g_ba)
                    out_buf[slot, jj, p, q] = top + wy[p] * (bot - top)

        def out_slice(bi):
            return out_hbm.at[pl.ds(roi_base + bi * NB, NB), :, :, c_g]

        def batch_body(bi, _):
            slot = bi & 1

            @pl.when((bi >= 2) & (slot == 0))
            def _():
                pltpu.make_async_copy(
                    out_buf.at[0], out_slice(bi - 2), sem0).wait()

            @pl.when((bi >= 2) & (slot == 1))
            def _():
                pltpu.make_async_copy(
                    out_buf.at[1], out_slice(bi - 2), sem1).wait()

            @plsc.parallel_loop(0, NB, unroll=2)
            def _(jj):
                fill_one(bi * NB + jj, slot, jj)

            @pl.when(slot == 0)
            def _():
                pltpu.make_async_copy(
                    out_buf.at[0], out_slice(bi), sem0).start()

            @pl.when(slot == 1)
            def _():
                pltpu.make_async_copy(
                    out_buf.at[1], out_slice(bi), sem1).start()

            return 0

        lax.fori_loop(0, n_batches, batch_body, 0)

        for bi in (n_batches - 2, n_batches - 1):
            slot = bi & 1
            pltpu.make_async_copy(
                out_buf.at[slot], out_slice(bi),
                sem0 if slot == 0 else sem1).wait()

    return roi_kernel


@jax.jit
def kernel(feature_map, rois):
    _, H, W, C = feature_map.shape
    _, R, _ = rois.shape
    fm_t = feature_map[0].reshape(H, W, NUM_C_GROUPS, C_PER_TILE)
    fm_t = jnp.transpose(fm_t, (2, 0, 1, 3))
    halves = []
    Rh = R // 2
    half_kernel = _make_roi_kernel(H, W, Rh, C)
    for h in range(2):
        rois_h = rois[0, h * Rh:(h + 1) * Rh]
        rois_t = rois_h.reshape(NUM_ROI_GROUPS, Rh // NUM_ROI_GROUPS, 4)
        rois_t = jnp.pad(rois_t, ((0, 0), (0, 0), (0, 12)))
        out = half_kernel(fm_t, rois_t)
        halves.append(out.reshape(1, Rh, POOL_H, POOL_W, C))
    return jnp.concatenate(halves, axis=1)


# speedup vs baseline: 1.9088x; 1.1108x over previous
import functools

import jax
import jax.numpy as jnp
from jax import lax
from jax.experimental import pallas as pl
from jax.experimental.pallas import tpu as pltpu
from jax.experimental.pallas import tpu_sc as plsc

POOL_H = 7
POOL_W = 7
NUM_ROI_GROUPS = 8
NUM_C_GROUPS = 4
C_PER_TILE = 16
NB = 25


def _make_roi_kernel(H, W, R, C):
    rois_per_tile = R // NUM_ROI_GROUPS
    n_batches = rois_per_tile // NB
    mesh = plsc.VectorSubcoreMesh(core_axis_name="c", subcore_axis_name="s")

    @functools.partial(
        pl.kernel,
        out_type=jax.ShapeDtypeStruct(
            (R, POOL_H, POOL_W, NUM_C_GROUPS, C_PER_TILE), jnp.float32),
        mesh=mesh,
        compiler_params=pltpu.CompilerParams(use_tc_tiling_on_sc=False),
        scratch_types=[
            pltpu.VMEM((H, W, C_PER_TILE), jnp.float32),
            pltpu.VMEM((rois_per_tile, 16), jnp.int32),
            pltpu.VMEM((2, NB, POOL_H, POOL_W, C_PER_TILE), jnp.float32),
            pltpu.SemaphoreType.DMA,
            pltpu.SemaphoreType.DMA,
        ],
    )
    def roi_kernel(fm_hbm, rois_hbm, out_hbm, fm_v, roi_v, out_buf,
                   sem0, sem1):
        wid = lax.axis_index("s") * 2 + lax.axis_index("c")
        c_g = wid // NUM_ROI_GROUPS
        roi_g = wid % NUM_ROI_GROUPS
        roi_base = roi_g * rois_per_tile

        pltpu.sync_copy(fm_hbm.at[c_g], fm_v)
        pltpu.sync_copy(rois_hbm.at[roi_g], roi_v)

        lane = lax.iota(jnp.int32, 16)
        pos = lane & 7
        posf = pos.astype(jnp.float32) + 0.5
        zeros = lane & 0
        idx_base = 1 - (lane >> 3)
        idx_end = idx_base + 2
        gather_dnums = lax.GatherDimensionNumbers(
            offset_dims=(), collapsed_slice_dims=(0,), start_index_map=(0,))

        def vgather(x, idx):
            return lax.gather(
                x, idx[:, None], gather_dnums, (1,),
                mode=lax.GatherScatterMode.PROMISE_IN_BOUNDS)

        def fill_one(j, slot, jj):
            rv = roi_v[j]
            base_s = vgather(rv, idx_base)
            ends = vgather(rv, idx_end)
            cl = ends - base_s
            clf = cl.astype(jnp.float32)
            coord = posf * (clf / float(POOL_H)) - 0.5
            coord = jnp.clip(coord, 0.0, jnp.maximum(clf - 1.0, 0.0))
            f0 = coord.astype(jnp.int32)
            w = coord - f0.astype(jnp.float32)
            a = base_s + f0
            y2m1 = rv[3] - 1
            x2m1 = rv[2] - 1
            ya = [a[p] for p in range(POOL_H)]
            xa = [a[8 + q] for q in range(POOL_W)]
            yb = [jnp.minimum(ya[p] + 1, y2m1) for p in range(POOL_H)]
            xb = [jnp.minimum(xa[q] + 1, x2m1) for q in range(POOL_W)]
            wy = [vgather(w, zeros + p) for p in range(POOL_H)]
            wx = [vgather(w, zeros + (8 + q)) for q in range(POOL_W)]
            for p in range(POOL_H):
                for q in range(POOL_W):
                    g_aa = fm_v[ya[p], xa[q]]
                    g_ab = fm_v[ya[p], xb[q]]
                    g_ba = fm_v[yb[p], xa[q]]
                    g_bb = fm_v[yb[p], xb[q]]
                    top = g_aa + wx[q] * (g_ab - g_aa)
                    bot = g_ba + wx[q] * (g_bb - g_ba)
                    out_buf[slot, jj, p, q] = top + wy[p] * (bot - top)

        def out_slice(bi):
            return out_hbm.at[pl.ds(roi_base + bi * NB, NB), :, :, c_g]

        def batch_body(bi, _):
            slot = bi & 1

            @pl.when((bi >= 2) & (slot == 0))
            def _():
                pltpu.make_async_copy(
                    out_buf.at[0], out_slice(bi - 2), sem0).wait()

            @pl.when((bi >= 2) & (slot == 1))
            def _():
                pltpu.make_async_copy(
                    out_buf.at[1], out_slice(bi - 2), sem1).wait()

            @plsc.parallel_loop(0, NB, unroll=2)
            def _(jj):
                fill_one(bi * NB + jj, slot, jj)

            @pl.when(slot == 0)
            def _():
                pltpu.make_async_copy(
                    out_buf.at[0], out_slice(bi), sem0).start()

            @pl.when(slot == 1)
            def _():
                pltpu.make_async_copy(
                    out_buf.at[1], out_slice(bi), sem1).start()

            return 0

        lax.fori_loop(0, n_batches, batch_body, 0)

        for bi in (n_batches - 2, n_batches - 1):
            slot = bi & 1
            pltpu.make_async_copy(
                out_buf.at[slot], out_slice(bi),
                sem0 if slot == 0 else sem1).wait()

    return roi_kernel


@jax.jit
def kernel(feature_map, rois):
    _, H, W, C = feature_map.shape
    _, R, _ = rois.shape
    fm_t = feature_map[0].reshape(H, W, NUM_C_GROUPS, C_PER_TILE)
    fm_t = jnp.transpose(fm_t, (2, 0, 1, 3))
    n_chunks = 5
    chunks = []
    Rc = R // n_chunks
    chunk_kernel = _make_roi_kernel(H, W, Rc, C)
    for h in range(n_chunks):
        rois_h = rois[0, h * Rc:(h + 1) * Rc]
        rois_t = rois_h.reshape(NUM_ROI_GROUPS, Rc // NUM_ROI_GROUPS, 4)
        rois_t = jnp.pad(rois_t, ((0, 0), (0, 0), (0, 12)))
        out = chunk_kernel(fm_t, rois_t)
        chunks.append(out.reshape(1, Rc, POOL_H, POOL_W, C))
    return jnp.concatenate(chunks, axis=1)
